# baseline (speedup 1.0000x reference)
import functools

import jax
import jax.numpy as jnp
from jax import lax
from jax.experimental import pallas as pl
from jax.experimental.pallas import tpu as pltpu
from jax.experimental.pallas import tpu_sc as plsc

_NC = 2
_NS = 16
_NWK = _NC * _NS
_R = 64


def _sc_body(b_total, m, w_hbm, p_hbm, prev_hbm, link_hbm, prec_hbm,
             w_v, p_v, in_buf, out_buf, prec_buf):
    wid = lax.axis_index("s") * _NC + lax.axis_index("c")
    bpw = b_total // _NWK
    nchunk = m // _R
    nvec = m // 16

    def batch_body(bi, _):
        b = wid * bpw + bi
        pltpu.sync_copy(w_hbm.at[b], w_v)
        pltpu.sync_copy(p_hbm.at[b], p_v)

        def chunk_body(c, _):
            pltpu.sync_copy(prev_hbm.at[b, pl.ds(c * _R, _R)], in_buf)

            def row_body(i, _):
                gi = c * _R + i
                base = (gi // 16) * 16
                lane = gi - base
                wrow = w_v[pl.ds(base, 16)]
                lanev = jnp.zeros((16,), jnp.int32) + lane
                wi = wrow.at[lanev].get(mode="promise_in_bounds")
                for j in range(nvec):
                    sl = pl.ds(j * 16, 16)
                    val = (1.0 - wi - w_v[sl]) * in_buf[i, sl] + wi * p_v[sl]
                    jj = lax.iota(jnp.int32, 16) + (j * 16)
                    out_buf[i, sl] = jnp.where(jj == gi, 0.0, val)
                return 0

            lax.fori_loop(0, _R, row_body, 0)
            pltpu.sync_copy(out_buf, link_hbm.at[b, pl.ds(c * _R, _R)])
            return 0

        lax.fori_loop(0, nchunk, chunk_body, 0)

        def sum_body(j, acc):
            return acc + w_v[pl.ds(j * 16, 16)]

        acc = lax.fori_loop(0, nvec, sum_body, jnp.zeros((16,), jnp.float32))
        ii16 = lax.iota(jnp.int32, 16)
        for s in (8, 4, 2, 1):
            acc = acc + acc.at[ii16 ^ s].get(mode="promise_in_bounds")
        coef = 1.0 - acc
        for j in range(nvec):
            sl = pl.ds(j * 16, 16)
            prec_buf[sl] = coef * p_v[sl] + w_v[sl]
        pltpu.sync_copy(prec_buf, prec_hbm.at[b])
        return 0

    lax.fori_loop(0, bpw, batch_body, 0)


def kernel(write_weights, prev_link, precedence_weights):
    b, nw, m = write_weights.shape
    w2 = write_weights.reshape(b, m)
    p2 = precedence_weights.reshape(b, m)
    prev3 = prev_link.reshape(b, m, m)

    mesh = plsc.VectorSubcoreMesh(core_axis_name="c", subcore_axis_name="s")
    sc = pl.kernel(
        functools.partial(_sc_body, b, m),
        mesh=mesh,
        out_type=[
            jax.ShapeDtypeStruct((b, m, m), jnp.float32),
            jax.ShapeDtypeStruct((b, m), jnp.float32),
        ],
        scratch_types=[
            pltpu.VMEM((m,), jnp.float32),
            pltpu.VMEM((m,), jnp.float32),
            pltpu.VMEM((_R, m), jnp.float32),
            pltpu.VMEM((_R, m), jnp.float32),
            pltpu.VMEM((m,), jnp.float32),
        ],
    )
    link3, prec2 = sc(w2, p2, prev3)
    return (link3.reshape(b, nw, m, m), prec2.reshape(b, nw, m))


# speedup vs baseline: 7.4694x; 7.4694x over previous
import jax
import jax.numpy as jnp
from jax import lax
from jax.experimental import pallas as pl
from jax.experimental.pallas import tpu as pltpu


def _linkage_body(w_ref, p_ref, prev_ref, link_ref, prec_ref):
    w = w_ref[:, 0]
    p = p_ref[:, 0]
    prev = prev_ref[:, 0]

    bb, m, _ = prev.shape
    wi = w[:, :, None]
    wj = w[:, None, :]
    link = (1.0 - wi - wj) * prev + wi * p[:, None, :]

    ii = lax.broadcasted_iota(jnp.int32, (m, m), 0)
    jj = lax.broadcasted_iota(jnp.int32, (m, m), 1)
    link = jnp.where((ii == jj)[None], 0.0, link)
    link_ref[:, 0] = link

    prec_ref[:, 0] = (1.0 - jnp.sum(w, axis=-1, keepdims=True)) * p + w


def kernel(write_weights, prev_link, precedence_weights):
    b, nw, m = write_weights.shape

    bb = 8
    grid = (b // bb,)
    vec_spec = pl.BlockSpec((bb, nw, m), lambda i: (i, 0, 0))
    mat_spec = pl.BlockSpec((bb, nw, m, m), lambda i: (i, 0, 0, 0))

    link, new_prec = pl.pallas_call(
        _linkage_body,
        grid=grid,
        in_specs=[vec_spec, vec_spec, mat_spec],
        out_specs=[mat_spec, vec_spec],
        out_shape=[
            jax.ShapeDtypeStruct(prev_link.shape, prev_link.dtype),
            jax.ShapeDtypeStruct(precedence_weights.shape, precedence_weights.dtype),
        ],
        compiler_params=pltpu.CompilerParams(
            dimension_semantics=("parallel",),
            vmem_limit_bytes=100 * 1024 * 1024,
        ),
    )(write_weights, precedence_weights, prev_link)
    return (link, new_prec)
